# baseline (speedup 1.0000x reference)
import functools

import jax
import jax.numpy as jnp
from jax import lax
from jax.experimental import pallas as pl
from jax.experimental.pallas import tpu as pltpu
from jax.experimental.pallas import tpu_sc as plsc

_NUM_SYM = 14
_FFT = 4096
_N = 2
_TIN = 384
_TOUT = 448
_TS = 32

_NC = 2
_NS = 16
_NW = _NC * _NS


def _sc_body(x_hbm, p_hbm, o_hbm, row0_v, row1_v, buf_v):
    wid = lax.axis_index("s") * _NC + lax.axis_index("c")
    pltpu.sync_copy(p_hbm.at[0:_TS], row0_v.at[:, 0:128])
    pltpu.sync_copy(p_hbm.at[0:_TS], row0_v.at[:, 128:256])
    pltpu.sync_copy(p_hbm.at[_TS:2 * _TS], row1_v.at[:, 0:128])
    pltpu.sync_copy(p_hbm.at[_TS:2 * _TS], row1_v.at[:, 128:256])

    rows_per_w = x_hbm.shape[0] // _NW
    for r in range(rows_per_w):
        b = wid * rows_per_w + r
        pltpu.sync_copy(x_hbm.at[b, 0:64], buf_v.at[0:64])
        pltpu.sync_copy(buf_v.at[0:64], o_hbm.at[b, 0:64])
        pltpu.sync_copy(x_hbm.at[b, 64:192], buf_v)
        pltpu.sync_copy(buf_v, o_hbm.at[b, 96:224])
        pltpu.sync_copy(x_hbm.at[b, 192:320], buf_v)
        pltpu.sync_copy(buf_v, o_hbm.at[b, 224:352])
        pltpu.sync_copy(x_hbm.at[b, 320:384], buf_v.at[0:64])
        pltpu.sync_copy(buf_v.at[0:64], o_hbm.at[b, 384:448])
        pltpu.sync_copy(row0_v, o_hbm.at[b, 64:96])
        pltpu.sync_copy(row1_v, o_hbm.at[b, 352:384])


def kernel(inputs, pilots):
    b = inputs.shape[0]
    x = inputs.reshape(b, _TIN, 128, _N).transpose(0, 1, 3, 2).reshape(b, _TIN, _N * 128)
    p = pilots.reshape(64, 128)
    mesh = plsc.VectorSubcoreMesh(core_axis_name="c", subcore_axis_name="s")
    run = functools.partial(
        pl.kernel,
        mesh=mesh,
        out_type=jax.ShapeDtypeStruct((b, _TOUT, _N * 128), inputs.dtype),
        scratch_types=[
            pltpu.VMEM((_TS, 256), jnp.float32),
            pltpu.VMEM((_TS, 256), jnp.float32),
            pltpu.VMEM((128, 256), jnp.float32),
        ],
    )(_sc_body)
    out = run(x, p)
    return (out.reshape(b, _TOUT, _N, 128)
               .transpose(0, 1, 3, 2)
               .reshape(b, 1, 1, _NUM_SYM, _FFT, _N))


# speedup vs baseline: 1.0399x; 1.0399x over previous
import functools

import jax
import jax.numpy as jnp
from jax import lax
from jax.experimental import pallas as pl
from jax.experimental.pallas import tpu as pltpu
from jax.experimental.pallas import tpu_sc as plsc

_NUM_SYM = 14
_FFT = 4096
_N = 2
_TIN = 384
_TOUT = 448
_TS = 32

_NC = 2
_NS = 16
_NW = _NC * _NS

_CHUNKS = (
    (0, 64, 0),
    (64, 192, 96),
    (192, 320, 224),
    (320, 384, 384),
)


def _sc_body(x_hbm, p_hbm, o_hbm, row0_v, row1_v, bufs, sems):
    wid = lax.axis_index("s") * _NC + lax.axis_index("c")
    pltpu.sync_copy(p_hbm.at[0:_TS], row0_v.at[:, 0:128])
    pltpu.sync_copy(p_hbm.at[0:_TS], row0_v.at[:, 128:256])
    pltpu.sync_copy(p_hbm.at[_TS:2 * _TS], row1_v.at[:, 0:128])
    pltpu.sync_copy(p_hbm.at[_TS:2 * _TS], row1_v.at[:, 128:256])

    rows_per_w = x_hbm.shape[0] // _NW
    for r in range(rows_per_w):
        b = wid * rows_per_w + r
        reads = []
        for k, (i0, i1, _) in enumerate(_CHUNKS):
            n = i1 - i0
            reads.append(pltpu.async_copy(
                x_hbm.at[b, i0:i1], bufs[k].at[0:n], sems.at[k]))
        wp0 = pltpu.async_copy(row0_v, o_hbm.at[b, 64:96], sems.at[8])
        wp1 = pltpu.async_copy(row1_v, o_hbm.at[b, 352:384], sems.at[9])
        writes = []
        for k, (i0, i1, o0) in enumerate(_CHUNKS):
            n = i1 - i0
            reads[k].wait()
            writes.append(pltpu.async_copy(
                bufs[k].at[0:n], o_hbm.at[b, o0:o0 + n], sems.at[4 + k]))
        for w in writes:
            w.wait()
        wp0.wait()
        wp1.wait()


def kernel(inputs, pilots):
    b = inputs.shape[0]
    x = inputs.reshape(b, _TIN, 128, _N).transpose(0, 1, 3, 2).reshape(b, _TIN, _N * 128)
    p = pilots.reshape(64, 128)
    mesh = plsc.VectorSubcoreMesh(core_axis_name="c", subcore_axis_name="s")

    def body(x_hbm, p_hbm, o_hbm, row0_v, row1_v, b0, b1, b2, b3, sems):
        _sc_body(x_hbm, p_hbm, o_hbm, row0_v, row1_v, (b0, b1, b2, b3), sems)

    run = functools.partial(
        pl.kernel,
        mesh=mesh,
        out_type=jax.ShapeDtypeStruct((b, _TOUT, _N * 128), inputs.dtype),
        scratch_types=[
            pltpu.VMEM((_TS, 256), jnp.float32),
            pltpu.VMEM((_TS, 256), jnp.float32),
            pltpu.VMEM((64, 256), jnp.float32),
            pltpu.VMEM((128, 256), jnp.float32),
            pltpu.VMEM((128, 256), jnp.float32),
            pltpu.VMEM((64, 256), jnp.float32),
            pltpu.SemaphoreType.DMA((10,)),
        ],
    )(body)
    out = run(x, p)
    return (out.reshape(b, _TOUT, _N, 128)
               .transpose(0, 1, 3, 2)
               .reshape(b, 1, 1, _NUM_SYM, _FFT, _N))


# speedup vs baseline: 1.0477x; 1.0076x over previous
import functools

import jax
import jax.numpy as jnp
from jax import lax
from jax.experimental import pallas as pl
from jax.experimental.pallas import tpu as pltpu
from jax.experimental.pallas import tpu_sc as plsc

_NUM_SYM = 14
_FFT = 4096
_N = 2
_TIN = 384
_TOUT = 448
_TS = 32

_NC = 2
_NS = 16
_NW = _NC * _NS

_CH = 64
_NBUF = 6

_OMAP = (0, 96, 160, 224, 288, 384)


def _sc_body(x_hbm, p_hbm, o_hbm, row0_v, row1_v, bufs, sems):
    wid = lax.axis_index("s") * _NC + lax.axis_index("c")
    pltpu.sync_copy(p_hbm.at[0:_TS], row0_v.at[:, 0:128])
    pltpu.sync_copy(p_hbm.at[0:_TS], row0_v.at[:, 128:256])
    pltpu.sync_copy(p_hbm.at[_TS:2 * _TS], row1_v.at[:, 0:128])
    pltpu.sync_copy(p_hbm.at[_TS:2 * _TS], row1_v.at[:, 128:256])

    rows_per_w = x_hbm.shape[0] // _NW
    chunks = []
    for r in range(rows_per_w):
        b = wid * rows_per_w + r
        for k in range(_TIN // _CH):
            chunks.append((b, k * _CH, _OMAP[k]))

    lookahead = 3
    pending_w = [None] * _NBUF
    pending_r = [None] * len(chunks)
    pilot_w = []

    def _writeback(c):
        s = c % _NBUF
        b, _, o0 = chunks[c]
        pending_r[c].wait()
        pending_w[s] = pltpu.async_copy(
            bufs[s], o_hbm.at[b, o0:o0 + _CH], sems.at[_NBUF + s])

    for c, (b, i0, o0) in enumerate(chunks):
        s = c % _NBUF
        if pending_w[s] is not None:
            pending_w[s].wait()
        pending_r[c] = pltpu.async_copy(
            x_hbm.at[b, i0:i0 + _CH], bufs[s], sems.at[s])
        if i0 == 0:
            if pilot_w:
                for h in pilot_w:
                    h.wait()
                pilot_w = []
            pilot_w.append(pltpu.async_copy(
                row0_v, o_hbm.at[b, 64:96], sems.at[2 * _NBUF]))
            pilot_w.append(pltpu.async_copy(
                row1_v, o_hbm.at[b, 352:384], sems.at[2 * _NBUF + 1]))
        if c >= lookahead:
            _writeback(c - lookahead)
    for c in range(len(chunks) - lookahead, len(chunks)):
        _writeback(c)
    for h in pending_w:
        if h is not None:
            h.wait()
    for h in pilot_w:
        h.wait()


def kernel(inputs, pilots):
    b = inputs.shape[0]
    x = inputs.reshape(b, _TIN, 128, _N).transpose(0, 1, 3, 2).reshape(b, _TIN, _N * 128)
    p = pilots.reshape(64, 128)
    mesh = plsc.VectorSubcoreMesh(core_axis_name="c", subcore_axis_name="s")

    def body(x_hbm, p_hbm, o_hbm, row0_v, row1_v, b0, b1, b2, b3, b4, b5, sems):
        _sc_body(x_hbm, p_hbm, o_hbm, row0_v, row1_v,
                 (b0, b1, b2, b3, b4, b5), sems)

    run = functools.partial(
        pl.kernel,
        mesh=mesh,
        out_type=jax.ShapeDtypeStruct((b, _TOUT, _N * 128), inputs.dtype),
        scratch_types=[
            pltpu.VMEM((_TS, 256), jnp.float32),
            pltpu.VMEM((_TS, 256), jnp.float32),
        ] + [pltpu.VMEM((_CH, 256), jnp.float32)] * _NBUF + [
            pltpu.SemaphoreType.DMA((2 * _NBUF + 2,)),
        ],
    )(body)
    out = run(x, p)
    return (out.reshape(b, _TOUT, _N, 128)
               .transpose(0, 1, 3, 2)
               .reshape(b, 1, 1, _NUM_SYM, _FFT, _N))
